# baseline (speedup 1.0000x reference)
import jax
import jax.numpy as jnp
from jax.experimental import pallas as pl

_PLACEHOLDER = 265
_ROWS_PER_BLOCK = 1024


def _select_body(tok_ref, emb_ref, pe_ref, out_ref):
    tok = tok_ref[...]
    out_ref[...] = jnp.where(tok == _PLACEHOLDER, pe_ref[...], emb_ref[...])


def kernel(tokenized_text, embedded_text, placeholder_embedding):
    B, N, D = embedded_text.shape
    rows = B * N
    emb = embedded_text.reshape(rows, D)
    tok = tokenized_text.reshape(rows, 1)

    R = _ROWS_PER_BLOCK
    num_blocks = pl.cdiv(rows, R)

    out = pl.pallas_call(
        _select_body,
        grid=(num_blocks,),
        in_specs=[
            pl.BlockSpec((R, 1), lambda i: (i, 0)),
            pl.BlockSpec((R, D), lambda i: (i, 0)),
            pl.BlockSpec((1, D), lambda i: (0, 0)),
        ],
        out_specs=pl.BlockSpec((R, D), lambda i: (i, 0)),
        out_shape=jax.ShapeDtypeStruct((rows, D), embedded_text.dtype),
    )(tok, emb, placeholder_embedding)
    return out.reshape(B, N, D)


# speedup vs baseline: 1.0259x; 1.0259x over previous
import jax
import jax.numpy as jnp
from jax.experimental import pallas as pl
from jax.experimental.pallas import tpu as pltpu

_PLACEHOLDER = 265
_ROWS_PER_BLOCK = 1024


def _select_body(tok_ref, emb_ref, pe_ref, out_ref):
    tok = tok_ref[0]
    tok_col = jnp.transpose(tok, (1, 0))
    mask = tok_col == _PLACEHOLDER
    out_ref[...] = jnp.where(mask, pe_ref[...], emb_ref[...])


def kernel(tokenized_text, embedded_text, placeholder_embedding):
    B, N, D = embedded_text.shape
    rows = B * N
    R = _ROWS_PER_BLOCK
    num_blocks = rows // R
    emb = embedded_text.reshape(rows, D)
    tok = tokenized_text.reshape(num_blocks, 1, R)

    out = pl.pallas_call(
        _select_body,
        grid=(num_blocks,),
        in_specs=[
            pl.BlockSpec((1, 1, R), lambda i: (i, 0, 0)),
            pl.BlockSpec((R, D), lambda i: (i, 0)),
            pl.BlockSpec((1, D), lambda i: (0, 0)),
        ],
        out_specs=pl.BlockSpec((R, D), lambda i: (i, 0)),
        out_shape=jax.ShapeDtypeStruct((rows, D), embedded_text.dtype),
        compiler_params=pltpu.CompilerParams(
            dimension_semantics=("parallel",),
        ),
    )(tok, emb, placeholder_embedding)
    return out.reshape(B, N, D)


# speedup vs baseline: 1.6731x; 1.6308x over previous
import jax
import jax.numpy as jnp
from jax.experimental import pallas as pl
from jax.experimental.pallas import tpu as pltpu

_PLACEHOLDER = 265
_S = 8
_BB = 16


def _body(tok_ref, pe_ref, emb_ref, out_ref, bufs, in_sems, out_sems):
    B, N, D = emb_ref.shape
    num = B // _BB
    rounds = num // _S
    pe = pe_ref[...]

    def in_copy(t, s):
        return pltpu.make_async_copy(
            emb_ref.at[pl.ds(t * _BB, _BB)], bufs.at[s], in_sems.at[s])

    def out_copy(t, s):
        return pltpu.make_async_copy(
            bufs.at[s], out_ref.at[pl.ds(t * _BB, _BB)], out_sems.at[s])

    for s in range(_S):
        in_copy(s, s).start()

    def round_body(r, carry):
        for s in range(_S):
            t = r * _S + s
            in_copy(t, s).wait()
            buf = bufs.at[s]
            for b in range(_BB):
                tok_col = jnp.transpose(
                    tok_ref[pl.ds(t * _BB + b, 1), :], (1, 0))
                mask = tok_col == _PLACEHOLDER
                buf[b] = jnp.where(mask, pe, buf[b])
            out_copy(t, s).start()

            @pl.when(t + _S < num)
            def _():
                out_copy(t, s).wait()
                in_copy(t + _S, s).start()
        return carry

    jax.lax.fori_loop(0, rounds, round_body, 0)

    for s in range(_S):
        out_copy(num - _S + s, s).wait()


def kernel(tokenized_text, embedded_text, placeholder_embedding):
    B, N, D = embedded_text.shape

    return pl.pallas_call(
        _body,
        in_specs=[
            pl.BlockSpec(memory_space=pltpu.VMEM),
            pl.BlockSpec(memory_space=pltpu.VMEM),
            pl.BlockSpec(memory_space=pltpu.MemorySpace.HBM),
        ],
        out_specs=pl.BlockSpec(memory_space=pltpu.MemorySpace.HBM),
        out_shape=jax.ShapeDtypeStruct((B, N, D), embedded_text.dtype),
        scratch_shapes=[
            pltpu.VMEM((_S, _BB, N, D), embedded_text.dtype),
            pltpu.SemaphoreType.DMA((_S,)),
            pltpu.SemaphoreType.DMA((_S,)),
        ],
    )(tokenized_text, placeholder_embedding, embedded_text)
